# Initial kernel scaffold; baseline (speedup 1.0000x reference)
import jax
import jax.numpy as jnp
from jax.experimental import pallas as pl


def kernel(x, table, W):
    raise NotImplementedError("write your pallas kernel here")


# speedup vs baseline: 2.5580x; 2.5580x over previous
import functools

import jax
import jax.numpy as jnp
from jax import lax
from jax.experimental import pallas as pl
from jax.experimental.pallas import tpu as pltpu
from jax.experimental.pallas import tpu_sc as plsc

TILE = 16
DIM_OUT = 128
W0 = 30.0
VOCAB = TILE * TILE

BATCH = 4
SEQ = 147456
B = BATCH * SEQ
NROW = B // 128

NW = 32
BPW = B // NW
CH = 128
NCH = BPW // CH


def _m_body(t_ref, w_ref, m_ref):
    m_ref[...] = W0 * lax.dot_general(
        t_ref[...], w_ref[...],
        dimension_numbers=(((1,), (1,)), ((), ())),
        preferred_element_type=jnp.float32,
    )


def _fused_table(table, W):
    return pl.pallas_call(
        _m_body,
        out_shape=jax.ShapeDtypeStruct((VOCAB, DIM_OUT), jnp.float32),
    )(table, W)


def _idx_body(x_ref, idx_ref):
    v = x_ref[...].astype(jnp.float32)
    lane = lax.broadcasted_iota(jnp.int32, (1, 2 * 128), 1)
    pat = jnp.where(lane % 2 == 0, 1.0, float(TILE))
    w = v * pat
    jj = lax.broadcasted_iota(jnp.int32, (2 * 128, 128), 0)
    kk = lax.broadcasted_iota(jnp.int32, (2 * 128, 128), 1)
    sel = (jj // 2 == kk).astype(jnp.float32)
    idx_f = lax.dot_general(
        w, sel,
        dimension_numbers=(((1,), (0,)), ((), ())),
        preferred_element_type=jnp.float32,
    )
    idx_ref[...] = idx_f.astype(jnp.int32)


def _indices(xr):
    bs = 512
    return pl.pallas_call(
        _idx_body,
        grid=(NROW // bs,),
        in_specs=[pl.BlockSpec((bs, 2 * 128), lambda i: (i, 0))],
        out_specs=pl.BlockSpec((bs, 128), lambda i: (i, 0)),
        out_shape=jax.ShapeDtypeStruct((NROW, 128), jnp.int32),
    )(xr)


@functools.cache
def _sc_gather_kernel():
    @functools.partial(
        pl.kernel,
        mesh=plsc.VectorSubcoreMesh(
            core_axis_name="c", subcore_axis_name="s", num_cores=2
        ),
        out_type=jax.ShapeDtypeStruct((B, DIM_OUT), jnp.float32),
        scratch_types=[
            pltpu.VMEM((NCH, CH), jnp.int32),
            pltpu.VMEM((CH, DIM_OUT), jnp.float32),
            pltpu.SemaphoreType.DMA,
        ],
    )
    def _sc_gather(m_hbm, idx_hbm, out_hbm, idx_v, buf, sem):
        wid = lax.axis_index("s") * 2 + lax.axis_index("c")
        pltpu.sync_copy(idx_hbm.at[pl.ds(wid * NCH, NCH)], idx_v)

        def body(j, carry):
            pltpu.async_copy(m_hbm.at[idx_v.at[j]], buf, sem).wait()
            pltpu.sync_copy(buf, out_hbm.at[pl.ds(wid * BPW + j * CH, CH)])
            return carry

        lax.fori_loop(0, NCH, body, 0)

    return _sc_gather


def kernel(x, table, W):
    M = _fused_table(table, W)
    xr = x.reshape(NROW, 2 * 128)
    idx2 = _indices(xr)
    out = _sc_gather_kernel()(M, idx2)
    return out.reshape(BATCH, SEQ, DIM_OUT)


# speedup vs baseline: 2.5620x; 1.0016x over previous
import functools

import jax
import jax.numpy as jnp
from jax import lax
from jax.experimental import pallas as pl
from jax.experimental.pallas import tpu as pltpu
from jax.experimental.pallas import tpu_sc as plsc

TILE = 16
DIM_OUT = 128
W0 = 30.0
VOCAB = TILE * TILE

BATCH = 4
SEQ = 147456
B = BATCH * SEQ
NROW = B // 128

NW = 32
BPW = B // NW
CH = 128
NCH = BPW // CH


def _m_body(t_ref, w_ref, m_ref):
    m_ref[...] = W0 * lax.dot_general(
        t_ref[...], w_ref[...],
        dimension_numbers=(((1,), (1,)), ((), ())),
        preferred_element_type=jnp.float32,
    )


def _fused_table(table, W):
    return pl.pallas_call(
        _m_body,
        out_shape=jax.ShapeDtypeStruct((VOCAB, DIM_OUT), jnp.float32),
    )(table, W)


def _idx_body(x_ref, idx_ref):
    v = x_ref[...].astype(jnp.float32)
    lane = lax.broadcasted_iota(jnp.int32, (1, 2 * 128), 1)
    pat = jnp.where(lane % 2 == 0, 1.0, float(TILE))
    w = v * pat
    jj = lax.broadcasted_iota(jnp.int32, (2 * 128, 128), 0)
    kk = lax.broadcasted_iota(jnp.int32, (2 * 128, 128), 1)
    sel = (jj // 2 == kk).astype(jnp.float32)
    idx_f = lax.dot_general(
        w, sel,
        dimension_numbers=(((1,), (0,)), ((), ())),
        preferred_element_type=jnp.float32,
    )
    idx_ref[...] = idx_f.astype(jnp.int32)


def _indices(xr):
    bs = 512
    return pl.pallas_call(
        _idx_body,
        grid=(NROW // bs,),
        in_specs=[pl.BlockSpec((bs, 2 * 128), lambda i: (i, 0))],
        out_specs=pl.BlockSpec((bs, 128), lambda i: (i, 0)),
        out_shape=jax.ShapeDtypeStruct((NROW, 128), jnp.int32),
    )(xr)


NBUF = 3


@functools.cache
def _sc_gather_kernel():
    @functools.partial(
        pl.kernel,
        mesh=plsc.VectorSubcoreMesh(
            core_axis_name="c", subcore_axis_name="s", num_cores=2
        ),
        out_type=jax.ShapeDtypeStruct((B, DIM_OUT), jnp.float32),
        scratch_types=[
            pltpu.VMEM((NCH, CH), jnp.int32),
            *[pltpu.VMEM((CH, DIM_OUT), jnp.float32) for _ in range(NBUF)],
            *[pltpu.SemaphoreType.DMA for _ in range(2 * NBUF)],
        ],
    )
    def _sc_gather(m_hbm, idx_hbm, out_hbm, idx_v, *rest):
        bufs = rest[:NBUF]
        gsems = rest[NBUF:2 * NBUF]
        ssems = rest[2 * NBUF:]
        wid = lax.axis_index("s") * 2 + lax.axis_index("c")
        base = wid * BPW
        pltpu.sync_copy(idx_hbm.at[pl.ds(wid * NCH, NCH)], idx_v)

        for b in range(NBUF):
            pltpu.async_copy(m_hbm.at[idx_v.at[b]], bufs[b], gsems[b])

        def body(r, carry):
            for b in range(NBUF):
                jj = r * NBUF + b
                dst = out_hbm.at[pl.ds(base + jj * CH, CH)]
                pltpu.make_async_copy(m_hbm.at[idx_v.at[jj]],
                                      bufs[b], gsems[b]).wait()
                pltpu.async_copy(bufs[b], dst, ssems[b])
                pltpu.make_async_copy(bufs[b], dst, ssems[b]).wait()
                nxt = lax.rem(jj + NBUF, NCH)
                pltpu.async_copy(m_hbm.at[idx_v.at[nxt]], bufs[b], gsems[b])
            return carry

        lax.fori_loop(0, NCH // NBUF, body, 0)

        for b in range(NBUF):
            pltpu.make_async_copy(m_hbm.at[idx_v.at[b]],
                                  bufs[b], gsems[b]).wait()

    return _sc_gather


def kernel(x, table, W):
    M = _fused_table(table, W)
    xr = x.reshape(NROW, 2 * 128)
    idx2 = _indices(xr)
    out = _sc_gather_kernel()(M, idx2)
    return out.reshape(BATCH, SEQ, DIM_OUT)
